# baseline (speedup 1.0000x reference)
import functools

import jax
import jax.numpy as jnp
from jax import lax
from jax.experimental import pallas as pl
from jax.experimental.pallas import tpu as pltpu
from jax.experimental.pallas import tpu_sc as plsc

B = 128
N = 32768
KSEL = 3276
NV = N // 16
NW = 32
RPW = B // NW
UNROLL = 8


def _row_topk_sum(xv, yv, hist, tot, lanes, lane_off, ones_i, zeros_i):
    @plsc.parallel_loop(0, NV, unroll=UNROLL)
    def _(i):
        s0 = i * 16
        xv[pl.ds(s0, 16)] = jnp.abs(xv[pl.ds(s0, 16)] - yv[pl.ds(s0, 16)])

    P = jnp.int32(0)
    kr = jnp.int32(KSEL)
    for p in range(4):
        sh = 24 - 8 * p

        @plsc.parallel_loop(0, 256, unroll=UNROLL)
        def _(i):
            hist[pl.ds(i * 16, 16)] = zeros_i

        if p == 0:
            @plsc.parallel_loop(0, NV, unroll=UNROLL)
            def _(i):
                v = lax.bitcast_convert_type(xv[pl.ds(i * 16, 16)], jnp.int32)
                bucket = lax.shift_right_logical(v, 24)
                plsc.addupdate_scatter(hist, [lane_off + bucket], ones_i)
        else:
            himask = jnp.int32(-(1 << (sh + 8)))
            pvec = jnp.full((16,), P, jnp.int32)
            shv = jnp.full((16,), sh, jnp.int32)

            @plsc.parallel_loop(0, NV, unroll=UNROLL)
            def _(i):
                v = lax.bitcast_convert_type(xv[pl.ds(i * 16, 16)], jnp.int32)
                bucket = lax.shift_right_logical(v, shv) & 255
                m = (v & himask) == pvec
                plsc.addupdate_scatter(hist, [lane_off + bucket],
                                       ones_i, mask=m)

        def chunk_body(c, _):
            def lane_body(l, t):
                return t + hist[pl.ds(l * 256 + c * 16, 16)]
            tot[pl.ds(c * 16, 16)] = lax.fori_loop(0, 16, lane_body, zeros_i)
            return 0
        lax.fori_loop(0, 16, chunk_body, 0)

        krv = jnp.full((16,), kr, jnp.int32)

        def suf_body(j, carries):
            carry, cnt_ge, over = carries
            t = tot[pl.ds((15 - j) * 16, 16)]
            s = jnp.flip(jnp.cumsum(jnp.flip(t))) + carry
            cnt_ge = cnt_ge + jnp.sum(jnp.where(s >= krv, 1, 0))
            over = jnp.maximum(over, jnp.max(jnp.where(s < krv, s, 0)))
            return jnp.max(s), cnt_ge, over
        _c, cnt_ge, over = lax.fori_loop(
            0, 16, suf_body, (jnp.int32(0), jnp.int32(0), jnp.int32(0)))

        bstar = cnt_ge - 1
        kr = kr - over
        P = P | lax.shift_left(bstar, sh)

    tvec = lax.bitcast_convert_type(jnp.full((16,), P, jnp.int32), jnp.float32)
    tval = jnp.max(tvec)

    @plsc.parallel_loop(0, NV, unroll=UNROLL,
                        carry=(jnp.zeros((16,), jnp.float32), zeros_i))
    def fin_loop(i, c):
        acc, cnt = c
        v = xv[pl.ds(i * 16, 16)]
        gt = v > tvec
        acc = acc + jnp.where(gt, v, jnp.float32(0.0))
        cnt = cnt + jnp.where(gt, 1, 0)
        return acc, cnt
    acc, cnt = fin_loop
    return (jnp.sum(acc)
            + (jnp.int32(KSEL) - jnp.sum(cnt)).astype(jnp.float32) * tval)


def _tec_body(x_hbm, y_hbm, out_hbm, xv0, xv1, yv, hist, tot, outv):
    cid = lax.axis_index("c")
    sid = lax.axis_index("s")
    wid = sid * 2 + cid

    lanes = lax.iota(jnp.int32, 16)
    lane_off = lanes * 256
    ones_i = jnp.ones((16,), jnp.int32)
    zeros_i = jnp.zeros((16,), jnp.int32)

    total = jnp.zeros((16,), jnp.float32)
    for row in range(RPW):
        xv = xv0 if row % 2 == 0 else xv1
        r = wid * RPW + row
        pltpu.sync_copy(x_hbm.at[r], xv)
        pltpu.sync_copy(y_hbm.at[r], yv)
        row_sum = _row_topk_sum(xv, yv, hist, tot,
                                lanes, lane_off, ones_i, zeros_i)
        total = total + jnp.full((16,), row_sum, jnp.float32)

    outv[...] = total
    pltpu.sync_copy(outv, out_hbm.at[wid])


@jax.jit
def _topk_partials(x, y):
    mesh = plsc.VectorSubcoreMesh(core_axis_name="c", subcore_axis_name="s")
    run = pl.kernel(
        _tec_body,
        out_type=jax.ShapeDtypeStruct((NW, 16), jnp.float32),
        mesh=mesh,
        compiler_params=pltpu.CompilerParams(needs_layout_passes=False),
        scratch_types=[
            pltpu.VMEM((N,), jnp.float32),
            pltpu.VMEM((N,), jnp.float32),
            pltpu.VMEM((N,), jnp.float32),
            pltpu.VMEM((4096,), jnp.int32),
            pltpu.VMEM((256,), jnp.int32),
            pltpu.VMEM((16,), jnp.float32),
        ],
    )
    return run(x, y)


def kernel(x, y):
    partials = _topk_partials(x, y)
    return jnp.sum(partials[:, 0]) / jnp.float32(B * KSEL)


# speedup vs baseline: 1.0127x; 1.0127x over previous
import functools

import jax
import jax.numpy as jnp
from jax import lax
from jax.experimental import pallas as pl
from jax.experimental.pallas import tpu as pltpu
from jax.experimental.pallas import tpu_sc as plsc

B = 128
N = 32768
KSEL = 3276
NV = N // 16
NW = 32
RPW = B // NW
UNROLL = 8


def _row_topk_sum(xv, yv, hist, tot, lanes, lane_off, ones_i, zeros_i):
    @plsc.parallel_loop(0, NV, unroll=UNROLL)
    def _(i):
        s0 = i * 16
        xv[pl.ds(s0, 16)] = jnp.abs(xv[pl.ds(s0, 16)] - yv[pl.ds(s0, 16)])

    P = jnp.int32(0)
    kr = jnp.int32(KSEL)
    for p in range(4):
        sh = 24 - 8 * p

        @plsc.parallel_loop(0, 256, unroll=UNROLL)
        def _(i):
            hist[pl.ds(i * 16, 16)] = zeros_i

        if p == 0:
            @plsc.parallel_loop(0, NV, unroll=UNROLL)
            def _(i):
                v = lax.bitcast_convert_type(xv[pl.ds(i * 16, 16)], jnp.int32)
                bucket = lax.shift_right_logical(v, 24)
                plsc.addupdate_scatter(hist, [lane_off + bucket], ones_i)
        else:
            himask = jnp.int32(-(1 << (sh + 8)))
            pvec = jnp.full((16,), P, jnp.int32)
            shv = jnp.full((16,), sh, jnp.int32)

            @plsc.parallel_loop(0, NV, unroll=UNROLL)
            def _(i):
                v = lax.bitcast_convert_type(xv[pl.ds(i * 16, 16)], jnp.int32)
                bucket = lax.shift_right_logical(v, shv) & 255
                m = (v & himask) == pvec
                plsc.addupdate_scatter(hist, [lane_off + bucket],
                                       ones_i, mask=m)

        @plsc.parallel_loop(0, 16, unroll=2)
        def _(c):
            t = hist[pl.ds(c * 16, 16)]
            for l in range(1, 16):
                t = t + hist[pl.ds(l * 256 + c * 16, 16)]
            tot[pl.ds(c * 16, 16)] = t

        krv = jnp.full((16,), kr, jnp.int32)

        def suf_body(j, carries):
            carry, cnt_ge, over = carries
            t = tot[pl.ds((15 - j) * 16, 16)]
            s = jnp.flip(jnp.cumsum(jnp.flip(t))) + carry
            cnt_ge = cnt_ge + jnp.sum(jnp.where(s >= krv, 1, 0))
            over = jnp.maximum(over, jnp.max(jnp.where(s < krv, s, 0)))
            return carry + jnp.sum(t), cnt_ge, over
        _c, cnt_ge, over = lax.fori_loop(
            0, 16, suf_body, (jnp.int32(0), jnp.int32(0), jnp.int32(0)))

        bstar = cnt_ge - 1
        kr = kr - over
        P = P | lax.shift_left(bstar, sh)

    tvec = lax.bitcast_convert_type(jnp.full((16,), P, jnp.int32), jnp.float32)
    tval = jnp.max(tvec)

    @plsc.parallel_loop(0, NV, unroll=UNROLL,
                        carry=(jnp.zeros((16,), jnp.float32), zeros_i))
    def fin_loop(i, c):
        acc, cnt = c
        v = xv[pl.ds(i * 16, 16)]
        gt = v > tvec
        acc = acc + jnp.where(gt, v, jnp.float32(0.0))
        cnt = cnt + jnp.where(gt, 1, 0)
        return acc, cnt
    acc, cnt = fin_loop
    return (jnp.sum(acc)
            + (jnp.int32(KSEL) - jnp.sum(cnt)).astype(jnp.float32) * tval)


def _tec_body(x_hbm, y_hbm, out_hbm, xv0, xv1, yv, hist, tot, outv):
    cid = lax.axis_index("c")
    sid = lax.axis_index("s")
    wid = sid * 2 + cid

    lanes = lax.iota(jnp.int32, 16)
    lane_off = lanes * 256
    ones_i = jnp.ones((16,), jnp.int32)
    zeros_i = jnp.zeros((16,), jnp.int32)

    total = jnp.zeros((16,), jnp.float32)
    for row in range(RPW):
        xv = xv0 if row % 2 == 0 else xv1
        r = wid * RPW + row
        pltpu.sync_copy(x_hbm.at[r], xv)
        pltpu.sync_copy(y_hbm.at[r], yv)
        row_sum = _row_topk_sum(xv, yv, hist, tot,
                                lanes, lane_off, ones_i, zeros_i)
        total = total + jnp.full((16,), row_sum, jnp.float32)

    outv[...] = total
    pltpu.sync_copy(outv, out_hbm.at[wid])


@jax.jit
def _topk_partials(x, y):
    mesh = plsc.VectorSubcoreMesh(core_axis_name="c", subcore_axis_name="s")
    run = pl.kernel(
        _tec_body,
        out_type=jax.ShapeDtypeStruct((NW, 16), jnp.float32),
        mesh=mesh,
        compiler_params=pltpu.CompilerParams(needs_layout_passes=False),
        scratch_types=[
            pltpu.VMEM((N,), jnp.float32),
            pltpu.VMEM((N,), jnp.float32),
            pltpu.VMEM((N,), jnp.float32),
            pltpu.VMEM((4096,), jnp.int32),
            pltpu.VMEM((256,), jnp.int32),
            pltpu.VMEM((16,), jnp.float32),
        ],
    )
    return run(x, y)


def kernel(x, y):
    partials = _topk_partials(x, y)
    return jnp.sum(partials[:, 0]) / jnp.float32(B * KSEL)


# speedup vs baseline: 1.0587x; 1.0455x over previous
import functools

import jax
import jax.numpy as jnp
from jax import lax
from jax.experimental import pallas as pl
from jax.experimental.pallas import tpu as pltpu
from jax.experimental.pallas import tpu_sc as plsc

B = 128
N = 32768
KSEL = 3276
NV = N // 16
NW = 32
RPW = B // NW
UNROLL = 8


def _row_topk_sum(xv, yv, hist, tot, lanes, lane_off, ones_i, zeros_i):
    @plsc.parallel_loop(0, NV, unroll=UNROLL)
    def _(i):
        s0 = i * 16
        xv[pl.ds(s0, 16)] = jnp.abs(xv[pl.ds(s0, 16)] - yv[pl.ds(s0, 16)])

    def merge_and_pick(kr):
        @plsc.parallel_loop(0, 16, unroll=2)
        def _(c):
            t = hist[pl.ds(c * 16, 16)]
            for l in range(1, 16):
                t = t + hist[pl.ds(l * 256 + c * 16, 16)]
            tot[pl.ds(c * 16, 16)] = t

        krv = jnp.full((16,), kr, jnp.int32)

        def suf_body(j, carries):
            carry, cnt_ge, over = carries
            t = tot[pl.ds((15 - j) * 16, 16)]
            s = jnp.flip(jnp.cumsum(jnp.flip(t))) + carry
            cnt_ge = cnt_ge + jnp.sum(jnp.where(s >= krv, 1, 0))
            over = jnp.maximum(over, jnp.max(jnp.where(s < krv, s, 0)))
            return carry + jnp.sum(t), cnt_ge, over
        _c, cnt_ge, over = lax.fori_loop(
            0, 16, suf_body, (jnp.int32(0), jnp.int32(0), jnp.int32(0)))
        return cnt_ge - 1, over

    @plsc.parallel_loop(0, 256, unroll=UNROLL)
    def _(i):
        hist[pl.ds(i * 16, 16)] = zeros_i

    @plsc.parallel_loop(0, NV, unroll=UNROLL)
    def _(i):
        v = lax.bitcast_convert_type(xv[pl.ds(i * 16, 16)], jnp.int32)
        bucket = lax.shift_right_logical(v, 24)
        plsc.addupdate_scatter(hist, [lane_off + bucket], ones_i)

    bstar, over = merge_and_pick(jnp.int32(KSEL))
    P = lax.shift_left(bstar, 24)
    kr = jnp.int32(KSEL) - over

    def pass_body(p, carries):
        P, kr = carries
        sh = 24 - 8 * p
        himask = lax.shift_left(jnp.int32(-1), sh + 8)
        pvec = jnp.full((16,), P, jnp.int32)
        shv = jnp.full((16,), sh, jnp.int32)
        hmv = jnp.full((16,), himask, jnp.int32)

        @plsc.parallel_loop(0, 256, unroll=UNROLL)
        def _(i):
            hist[pl.ds(i * 16, 16)] = zeros_i

        @plsc.parallel_loop(0, NV, unroll=UNROLL)
        def _(i):
            v = lax.bitcast_convert_type(xv[pl.ds(i * 16, 16)], jnp.int32)
            bucket = lax.shift_right_logical(v, shv) & 255
            m = (v & hmv) == pvec
            plsc.addupdate_scatter(hist, [lane_off + bucket],
                                   ones_i, mask=m)

        bstar, over = merge_and_pick(kr)
        return P | lax.shift_left(bstar, sh), kr - over

    P, kr = lax.fori_loop(1, 4, pass_body, (P, kr))

    tvec = lax.bitcast_convert_type(jnp.full((16,), P, jnp.int32), jnp.float32)
    tval = jnp.max(tvec)

    @plsc.parallel_loop(0, NV, unroll=UNROLL,
                        carry=(jnp.zeros((16,), jnp.float32), zeros_i))
    def fin_loop(i, c):
        acc, cnt = c
        v = xv[pl.ds(i * 16, 16)]
        gt = v > tvec
        acc = acc + jnp.where(gt, v, jnp.float32(0.0))
        cnt = cnt + jnp.where(gt, 1, 0)
        return acc, cnt
    acc, cnt = fin_loop
    return (jnp.sum(acc)
            + (jnp.int32(KSEL) - jnp.sum(cnt)).astype(jnp.float32) * tval)


def _tec_body(x_hbm, y_hbm, out_hbm, xv0, xv1, yv, hist, tot, outv):
    cid = lax.axis_index("c")
    sid = lax.axis_index("s")
    wid = sid * 2 + cid

    lanes = lax.iota(jnp.int32, 16)
    lane_off = lanes * 256
    ones_i = jnp.ones((16,), jnp.int32)
    zeros_i = jnp.zeros((16,), jnp.int32)

    def pair_body(j, total):
        r = wid * RPW + 2 * j
        pltpu.sync_copy(x_hbm.at[r], xv0)
        pltpu.sync_copy(y_hbm.at[r], yv)
        s0 = _row_topk_sum(xv0, yv, hist, tot,
                           lanes, lane_off, ones_i, zeros_i)
        pltpu.sync_copy(x_hbm.at[r + 1], xv1)
        pltpu.sync_copy(y_hbm.at[r + 1], yv)
        s1 = _row_topk_sum(xv1, yv, hist, tot,
                           lanes, lane_off, ones_i, zeros_i)
        return total + jnp.full((16,), s0 + s1, jnp.float32)

    total = lax.fori_loop(0, RPW // 2, pair_body, jnp.zeros((16,), jnp.float32))

    outv[...] = total
    pltpu.sync_copy(outv, out_hbm.at[wid])


@jax.jit
def _topk_partials(x, y):
    mesh = plsc.VectorSubcoreMesh(core_axis_name="c", subcore_axis_name="s")
    run = pl.kernel(
        _tec_body,
        out_type=jax.ShapeDtypeStruct((NW, 16), jnp.float32),
        mesh=mesh,
        compiler_params=pltpu.CompilerParams(needs_layout_passes=False),
        scratch_types=[
            pltpu.VMEM((N,), jnp.float32),
            pltpu.VMEM((N,), jnp.float32),
            pltpu.VMEM((N,), jnp.float32),
            pltpu.VMEM((4096,), jnp.int32),
            pltpu.VMEM((256,), jnp.int32),
            pltpu.VMEM((16,), jnp.float32),
        ],
    )
    return run(x, y)


def kernel(x, y):
    partials = _topk_partials(x, y)
    return jnp.sum(partials[:, 0]) / jnp.float32(B * KSEL)


# speedup vs baseline: 1.1567x; 1.0926x over previous
import functools

import jax
import jax.numpy as jnp
from jax import lax
from jax.experimental import pallas as pl
from jax.experimental.pallas import tpu as pltpu
from jax.experimental.pallas import tpu_sc as plsc

B = 128
N = 32768
KSEL = 3276
NV = N // 16
NW = 32
RPW = B // NW
UNROLL = 8


def _row_topk_sum(xv, yv, hist, tot, lanes, lane_off, ones_i, zeros_i,
                  prefetch):
    @plsc.parallel_loop(0, NV, unroll=UNROLL)
    def _(i):
        s0 = i * 16
        xv[pl.ds(s0, 16)] = jnp.abs(xv[pl.ds(s0, 16)] - yv[pl.ds(s0, 16)])

    def merge_and_pick(kr):
        @plsc.parallel_loop(0, 16, unroll=2)
        def _(c):
            t = hist[pl.ds(c * 16, 16)]
            for l in range(1, 16):
                t = t + hist[pl.ds(l * 256 + c * 16, 16)]
            tot[pl.ds(c * 16, 16)] = t

        krv = jnp.full((16,), kr, jnp.int32)

        def suf_body(j, carries):
            carry, cnt_ge, over = carries
            t = tot[pl.ds((15 - j) * 16, 16)]
            s = jnp.flip(jnp.cumsum(jnp.flip(t))) + carry
            cnt_ge = cnt_ge + jnp.sum(jnp.where(s >= krv, 1, 0))
            over = jnp.maximum(over, jnp.max(jnp.where(s < krv, s, 0)))
            return carry + jnp.sum(t), cnt_ge, over
        _c, cnt_ge, over = lax.fori_loop(
            0, 16, suf_body, (jnp.int32(0), jnp.int32(0), jnp.int32(0)))
        return cnt_ge - 1, over

    @plsc.parallel_loop(0, 256, unroll=UNROLL)
    def _(i):
        hist[pl.ds(i * 16, 16)] = zeros_i

    @plsc.parallel_loop(0, NV, unroll=UNROLL)
    def _(i):
        v = lax.bitcast_convert_type(xv[pl.ds(i * 16, 16)], jnp.int32)
        bucket = lax.shift_right_logical(v, 24)
        plsc.addupdate_scatter(hist, [lane_off + bucket], ones_i)

    bstar, over = merge_and_pick(jnp.int32(KSEL))
    P = lax.shift_left(bstar, 24)
    kr = jnp.int32(KSEL) - over

    pvec1 = jnp.full((16,), P, jnp.int32)

    @plsc.parallel_loop(0, 256, unroll=UNROLL)
    def _(i):
        hist[pl.ds(i * 16, 16)] = zeros_i

    @plsc.parallel_loop(0, NV, unroll=UNROLL)
    def _(i):
        v = lax.bitcast_convert_type(xv[pl.ds(i * 16, 16)], jnp.int32)
        bucket = lax.shift_right_logical(v, 16) & 255
        m = (v & jnp.int32(-(1 << 24))) == pvec1
        plsc.addupdate_scatter(hist, [lane_off + bucket], ones_i, mask=m)

    prefetch()

    bstar, over = merge_and_pick(kr)
    P = P | lax.shift_left(bstar, 16)
    kr = kr - over

    def pass_body(p, carries):
        P, kr = carries
        sh = 24 - 8 * p
        himask = lax.shift_left(jnp.int32(-1), sh + 8)
        pvec = jnp.full((16,), P, jnp.int32)
        shv = jnp.full((16,), sh, jnp.int32)
        hmv = jnp.full((16,), himask, jnp.int32)

        @plsc.parallel_loop(0, 256, unroll=UNROLL)
        def _(i):
            hist[pl.ds(i * 16, 16)] = zeros_i

        @plsc.parallel_loop(0, NV, unroll=UNROLL)
        def _(i):
            v = lax.bitcast_convert_type(xv[pl.ds(i * 16, 16)], jnp.int32)
            bucket = lax.shift_right_logical(v, shv) & 255
            m = (v & hmv) == pvec
            plsc.addupdate_scatter(hist, [lane_off + bucket],
                                   ones_i, mask=m)

        bstar, over = merge_and_pick(kr)
        return P | lax.shift_left(bstar, sh), kr - over

    P, kr = lax.fori_loop(2, 4, pass_body, (P, kr))

    tvec = lax.bitcast_convert_type(jnp.full((16,), P, jnp.int32), jnp.float32)
    tval = jnp.max(tvec)

    @plsc.parallel_loop(0, NV, unroll=UNROLL,
                        carry=(jnp.zeros((16,), jnp.float32), zeros_i))
    def fin_loop(i, c):
        acc, cnt = c
        v = xv[pl.ds(i * 16, 16)]
        gt = v > tvec
        acc = acc + jnp.where(gt, v, jnp.float32(0.0))
        cnt = cnt + jnp.where(gt, 1, 0)
        return acc, cnt
    acc, cnt = fin_loop
    return (jnp.sum(acc)
            + (jnp.int32(KSEL) - jnp.sum(cnt)).astype(jnp.float32) * tval)


def _tec_body(x_hbm, y_hbm, out_hbm, xv0, xv1, yv, hist, tot, outv,
              semx, semy):
    cid = lax.axis_index("c")
    sid = lax.axis_index("s")
    wid = sid * 2 + cid

    lanes = lax.iota(jnp.int32, 16)
    lane_off = lanes * 256
    ones_i = jnp.ones((16,), jnp.int32)
    zeros_i = jnp.zeros((16,), jnp.int32)

    def issue(rr, xbuf):
        pltpu.async_copy(x_hbm.at[rr], xbuf, semx)
        pltpu.async_copy(y_hbm.at[rr], yv, semy)

    def drain():
        pltpu.make_async_copy(x_hbm.at[0], xv0, semx).wait()
        pltpu.make_async_copy(y_hbm.at[0], yv, semy).wait()

    issue(wid * RPW, xv0)

    def pair_body(j, total):
        r = wid * RPW + 2 * j
        drain()
        s0 = _row_topk_sum(xv0, yv, hist, tot,
                           lanes, lane_off, ones_i, zeros_i,
                           lambda: issue(r + 1, xv1))
        drain()
        s1 = _row_topk_sum(xv1, yv, hist, tot,
                           lanes, lane_off, ones_i, zeros_i,
                           lambda: issue(jnp.minimum(r + 2, B - 1), xv0))
        return total + jnp.full((16,), s0 + s1, jnp.float32)

    total = lax.fori_loop(0, RPW // 2, pair_body, jnp.zeros((16,), jnp.float32))
    drain()

    outv[...] = total
    sync_out = pltpu.sync_copy(outv, out_hbm.at[wid])


@jax.jit
def _topk_partials(x, y):
    mesh = plsc.VectorSubcoreMesh(core_axis_name="c", subcore_axis_name="s")
    run = pl.kernel(
        _tec_body,
        out_type=jax.ShapeDtypeStruct((NW, 16), jnp.float32),
        mesh=mesh,
        compiler_params=pltpu.CompilerParams(needs_layout_passes=False),
        scratch_types=[
            pltpu.VMEM((N,), jnp.float32),
            pltpu.VMEM((N,), jnp.float32),
            pltpu.VMEM((N,), jnp.float32),
            pltpu.VMEM((4096,), jnp.int32),
            pltpu.VMEM((256,), jnp.int32),
            pltpu.VMEM((16,), jnp.float32),
            pltpu.SemaphoreType.DMA,
            pltpu.SemaphoreType.DMA,
        ],
    )
    return run(x, y)


def kernel(x, y):
    partials = _topk_partials(x, y)
    return jnp.sum(partials[:, 0]) / jnp.float32(B * KSEL)


# speedup vs baseline: 1.1637x; 1.0060x over previous
import functools

import jax
import jax.numpy as jnp
from jax import lax
from jax.experimental import pallas as pl
from jax.experimental.pallas import tpu as pltpu
from jax.experimental.pallas import tpu_sc as plsc

B = 128
N = 32768
KSEL = 3276
NV = N // 16
NW = 32
RPW = B // NW
UNROLL = 16


def _row_topk_sum(xv, yv, hist, tot, lanes, lane_off, ones_i, zeros_i,
                  prefetch):
    @plsc.parallel_loop(0, NV, unroll=UNROLL)
    def _(i):
        s0 = i * 16
        xv[pl.ds(s0, 16)] = jnp.abs(xv[pl.ds(s0, 16)] - yv[pl.ds(s0, 16)])

    def merge_and_pick(kr):
        @plsc.parallel_loop(0, 16, unroll=2)
        def _(c):
            t = hist[pl.ds(c * 16, 16)]
            for l in range(1, 16):
                t = t + hist[pl.ds(l * 256 + c * 16, 16)]
            tot[pl.ds(c * 16, 16)] = t

        krv = jnp.full((16,), kr, jnp.int32)

        def suf_body(j, carries):
            carry, cnt_ge, over = carries
            t = tot[pl.ds((15 - j) * 16, 16)]
            s = jnp.flip(jnp.cumsum(jnp.flip(t))) + carry
            cnt_ge = cnt_ge + jnp.sum(jnp.where(s >= krv, 1, 0))
            over = jnp.maximum(over, jnp.max(jnp.where(s < krv, s, 0)))
            return carry + jnp.sum(t), cnt_ge, over
        _c, cnt_ge, over = lax.fori_loop(
            0, 16, suf_body, (jnp.int32(0), jnp.int32(0), jnp.int32(0)))
        return cnt_ge - 1, over

    @plsc.parallel_loop(0, 256, unroll=UNROLL)
    def _(i):
        hist[pl.ds(i * 16, 16)] = zeros_i

    @plsc.parallel_loop(0, NV, unroll=UNROLL)
    def _(i):
        v = lax.bitcast_convert_type(xv[pl.ds(i * 16, 16)], jnp.int32)
        bucket = lax.shift_right_logical(v, 24)
        plsc.addupdate_scatter(hist, [lane_off + bucket], ones_i)

    bstar, over = merge_and_pick(jnp.int32(KSEL))
    P = lax.shift_left(bstar, 24)
    kr = jnp.int32(KSEL) - over

    pvec1 = jnp.full((16,), P, jnp.int32)

    @plsc.parallel_loop(0, 256, unroll=UNROLL)
    def _(i):
        hist[pl.ds(i * 16, 16)] = zeros_i

    @plsc.parallel_loop(0, NV, unroll=UNROLL)
    def _(i):
        v = lax.bitcast_convert_type(xv[pl.ds(i * 16, 16)], jnp.int32)
        bucket = lax.shift_right_logical(v, 16) & 255
        m = (v & jnp.int32(-(1 << 24))) == pvec1
        plsc.addupdate_scatter(hist, [lane_off + bucket], ones_i, mask=m)

    prefetch()

    bstar, over = merge_and_pick(kr)
    P = P | lax.shift_left(bstar, 16)
    kr = kr - over

    def pass_body(p, carries):
        P, kr = carries
        sh = 24 - 8 * p
        himask = lax.shift_left(jnp.int32(-1), sh + 8)
        pvec = jnp.full((16,), P, jnp.int32)
        shv = jnp.full((16,), sh, jnp.int32)
        hmv = jnp.full((16,), himask, jnp.int32)

        @plsc.parallel_loop(0, 256, unroll=UNROLL)
        def _(i):
            hist[pl.ds(i * 16, 16)] = zeros_i

        @plsc.parallel_loop(0, NV, unroll=UNROLL)
        def _(i):
            v = lax.bitcast_convert_type(xv[pl.ds(i * 16, 16)], jnp.int32)
            bucket = lax.shift_right_logical(v, shv) & 255
            m = (v & hmv) == pvec
            plsc.addupdate_scatter(hist, [lane_off + bucket],
                                   ones_i, mask=m)

        bstar, over = merge_and_pick(kr)
        return P | lax.shift_left(bstar, sh), kr - over

    P, kr = lax.fori_loop(2, 4, pass_body, (P, kr))

    tvec = lax.bitcast_convert_type(jnp.full((16,), P, jnp.int32), jnp.float32)
    tval = jnp.max(tvec)

    @plsc.parallel_loop(0, NV, unroll=UNROLL,
                        carry=(jnp.zeros((16,), jnp.float32), zeros_i))
    def fin_loop(i, c):
        acc, cnt = c
        v = xv[pl.ds(i * 16, 16)]
        gt = v > tvec
        acc = acc + jnp.where(gt, v, jnp.float32(0.0))
        cnt = cnt + jnp.where(gt, 1, 0)
        return acc, cnt
    acc, cnt = fin_loop
    return (jnp.sum(acc)
            + (jnp.int32(KSEL) - jnp.sum(cnt)).astype(jnp.float32) * tval)


def _tec_body(x_hbm, y_hbm, out_hbm, xv0, xv1, yv, hist, tot, outv,
              semx, semy):
    cid = lax.axis_index("c")
    sid = lax.axis_index("s")
    wid = sid * 2 + cid

    lanes = lax.iota(jnp.int32, 16)
    lane_off = lanes * 256
    ones_i = jnp.ones((16,), jnp.int32)
    zeros_i = jnp.zeros((16,), jnp.int32)

    def issue(rr, xbuf):
        pltpu.async_copy(x_hbm.at[rr], xbuf, semx)
        pltpu.async_copy(y_hbm.at[rr], yv, semy)

    def drain():
        pltpu.make_async_copy(x_hbm.at[0], xv0, semx).wait()
        pltpu.make_async_copy(y_hbm.at[0], yv, semy).wait()

    issue(wid * RPW, xv0)

    def pair_body(j, total):
        r = wid * RPW + 2 * j
        drain()
        s0 = _row_topk_sum(xv0, yv, hist, tot,
                           lanes, lane_off, ones_i, zeros_i,
                           lambda: issue(r + 1, xv1))
        drain()
        s1 = _row_topk_sum(xv1, yv, hist, tot,
                           lanes, lane_off, ones_i, zeros_i,
                           lambda: issue(jnp.minimum(r + 2, B - 1), xv0))
        return total + jnp.full((16,), s0 + s1, jnp.float32)

    total = lax.fori_loop(0, RPW // 2, pair_body, jnp.zeros((16,), jnp.float32))
    drain()

    outv[...] = total
    sync_out = pltpu.sync_copy(outv, out_hbm.at[wid])


@jax.jit
def _topk_partials(x, y):
    mesh = plsc.VectorSubcoreMesh(core_axis_name="c", subcore_axis_name="s")
    run = pl.kernel(
        _tec_body,
        out_type=jax.ShapeDtypeStruct((NW, 16), jnp.float32),
        mesh=mesh,
        compiler_params=pltpu.CompilerParams(needs_layout_passes=False),
        scratch_types=[
            pltpu.VMEM((N,), jnp.float32),
            pltpu.VMEM((N,), jnp.float32),
            pltpu.VMEM((N,), jnp.float32),
            pltpu.VMEM((4096,), jnp.int32),
            pltpu.VMEM((256,), jnp.int32),
            pltpu.VMEM((16,), jnp.float32),
            pltpu.SemaphoreType.DMA,
            pltpu.SemaphoreType.DMA,
        ],
    )
    return run(x, y)


def kernel(x, y):
    partials = _topk_partials(x, y)
    return jnp.sum(partials[:, 0]) / jnp.float32(B * KSEL)


# speedup vs baseline: 1.1777x; 1.0121x over previous
import functools

import jax
import jax.numpy as jnp
from jax import lax
from jax.experimental import pallas as pl
from jax.experimental.pallas import tpu as pltpu
from jax.experimental.pallas import tpu_sc as plsc

B = 128
N = 32768
KSEL = 3276
NV = N // 16
NW = 32
RPW = B // NW
UNROLL = 16


def _row_topk_sum(xv, yv, hist, tot, lanes, lane_off, ones_i, zeros_i,
                  prefetch):
    @plsc.parallel_loop(0, NV, unroll=UNROLL)
    def _(i):
        s0 = i * 16
        xv[pl.ds(s0, 16)] = jnp.abs(xv[pl.ds(s0, 16)] - yv[pl.ds(s0, 16)])

    def merge_and_pick(kr):
        @plsc.parallel_loop(0, 16, unroll=2)
        def _(c):
            t = hist[pl.ds(c * 16, 16)]
            for l in range(1, 16):
                t = t + hist[pl.ds(l * 256 + c * 16, 16)]
            tot[pl.ds(c * 16, 16)] = jnp.flip(jnp.cumsum(jnp.flip(t)))

        krv = jnp.full((16,), kr, jnp.int32)
        ct = plsc.load_gather(tot, [lanes * 16])
        sc = jnp.flip(jnp.cumsum(jnp.flip(ct)))
        above = jnp.max(jnp.where(sc < krv, sc, 0))
        cstar = jnp.sum(jnp.where(sc >= krv, 1, 0)) - 1
        s_in = tot[pl.ds(cstar * 16, 16)] + above
        cnt_in = jnp.sum(jnp.where(s_in >= krv, 1, 0))
        bstar = cstar * 16 + cnt_in - 1
        over = jnp.maximum(above, jnp.max(jnp.where(s_in < krv, s_in, 0)))
        return bstar, over

    @plsc.parallel_loop(0, 256, unroll=UNROLL)
    def _(i):
        hist[pl.ds(i * 16, 16)] = zeros_i

    @plsc.parallel_loop(0, NV, unroll=UNROLL)
    def _(i):
        v = lax.bitcast_convert_type(xv[pl.ds(i * 16, 16)], jnp.int32)
        bucket = lax.shift_right_logical(v, 24)
        plsc.addupdate_scatter(hist, [lane_off + bucket], ones_i)

    bstar, over = merge_and_pick(jnp.int32(KSEL))
    P = lax.shift_left(bstar, 24)
    kr = jnp.int32(KSEL) - over

    pvec1 = jnp.full((16,), P, jnp.int32)

    @plsc.parallel_loop(0, 256, unroll=UNROLL)
    def _(i):
        hist[pl.ds(i * 16, 16)] = zeros_i

    @plsc.parallel_loop(0, NV, unroll=UNROLL)
    def _(i):
        v = lax.bitcast_convert_type(xv[pl.ds(i * 16, 16)], jnp.int32)
        bucket = lax.shift_right_logical(v, 16) & 255
        m = (v & jnp.int32(-(1 << 24))) == pvec1
        plsc.addupdate_scatter(hist, [lane_off + bucket], ones_i, mask=m)

    prefetch()

    bstar, over = merge_and_pick(kr)
    P = P | lax.shift_left(bstar, 16)
    kr = kr - over

    def pass_body(p, carries):
        P, kr = carries
        sh = 24 - 8 * p
        himask = lax.shift_left(jnp.int32(-1), sh + 8)
        pvec = jnp.full((16,), P, jnp.int32)
        shv = jnp.full((16,), sh, jnp.int32)
        hmv = jnp.full((16,), himask, jnp.int32)

        @plsc.parallel_loop(0, 256, unroll=UNROLL)
        def _(i):
            hist[pl.ds(i * 16, 16)] = zeros_i

        @plsc.parallel_loop(0, NV, unroll=UNROLL)
        def _(i):
            v = lax.bitcast_convert_type(xv[pl.ds(i * 16, 16)], jnp.int32)
            bucket = lax.shift_right_logical(v, shv) & 255
            m = (v & hmv) == pvec
            plsc.addupdate_scatter(hist, [lane_off + bucket],
                                   ones_i, mask=m)

        bstar, over = merge_and_pick(kr)
        return P | lax.shift_left(bstar, sh), kr - over

    P, kr = lax.fori_loop(2, 4, pass_body, (P, kr))

    tvec = lax.bitcast_convert_type(jnp.full((16,), P, jnp.int32), jnp.float32)
    tval = jnp.max(tvec)

    @plsc.parallel_loop(0, NV, unroll=UNROLL,
                        carry=(jnp.zeros((16,), jnp.float32), zeros_i))
    def fin_loop(i, c):
        acc, cnt = c
        v = xv[pl.ds(i * 16, 16)]
        gt = v > tvec
        acc = acc + jnp.where(gt, v, jnp.float32(0.0))
        cnt = cnt + jnp.where(gt, 1, 0)
        return acc, cnt
    acc, cnt = fin_loop
    return (jnp.sum(acc)
            + (jnp.int32(KSEL) - jnp.sum(cnt)).astype(jnp.float32) * tval)


def _tec_body(x_hbm, y_hbm, out_hbm, xv0, xv1, yv, hist, tot, outv,
              semx, semy):
    cid = lax.axis_index("c")
    sid = lax.axis_index("s")
    wid = sid * 2 + cid

    lanes = lax.iota(jnp.int32, 16)
    lane_off = lanes * 256
    ones_i = jnp.ones((16,), jnp.int32)
    zeros_i = jnp.zeros((16,), jnp.int32)

    def issue(rr, xbuf):
        pltpu.async_copy(x_hbm.at[rr], xbuf, semx)
        pltpu.async_copy(y_hbm.at[rr], yv, semy)

    def drain():
        pltpu.make_async_copy(x_hbm.at[0], xv0, semx).wait()
        pltpu.make_async_copy(y_hbm.at[0], yv, semy).wait()

    issue(wid * RPW, xv0)

    def pair_body(j, total):
        r = wid * RPW + 2 * j
        drain()
        s0 = _row_topk_sum(xv0, yv, hist, tot,
                           lanes, lane_off, ones_i, zeros_i,
                           lambda: issue(r + 1, xv1))
        drain()
        s1 = _row_topk_sum(xv1, yv, hist, tot,
                           lanes, lane_off, ones_i, zeros_i,
                           lambda: issue(jnp.minimum(r + 2, B - 1), xv0))
        return total + jnp.full((16,), s0 + s1, jnp.float32)

    total = lax.fori_loop(0, RPW // 2, pair_body, jnp.zeros((16,), jnp.float32))
    drain()

    outv[...] = total
    sync_out = pltpu.sync_copy(outv, out_hbm.at[wid])


@jax.jit
def _topk_partials(x, y):
    mesh = plsc.VectorSubcoreMesh(core_axis_name="c", subcore_axis_name="s")
    run = pl.kernel(
        _tec_body,
        out_type=jax.ShapeDtypeStruct((NW, 16), jnp.float32),
        mesh=mesh,
        compiler_params=pltpu.CompilerParams(needs_layout_passes=False),
        scratch_types=[
            pltpu.VMEM((N,), jnp.float32),
            pltpu.VMEM((N,), jnp.float32),
            pltpu.VMEM((N,), jnp.float32),
            pltpu.VMEM((4096,), jnp.int32),
            pltpu.VMEM((256,), jnp.int32),
            pltpu.VMEM((16,), jnp.float32),
            pltpu.SemaphoreType.DMA,
            pltpu.SemaphoreType.DMA,
        ],
    )
    return run(x, y)


def kernel(x, y):
    partials = _topk_partials(x, y)
    return jnp.sum(partials[:, 0]) / jnp.float32(B * KSEL)


# speedup vs baseline: 1.2214x; 1.0371x over previous
import functools

import jax
import jax.numpy as jnp
from jax import lax
from jax.experimental import pallas as pl
from jax.experimental.pallas import tpu as pltpu
from jax.experimental.pallas import tpu_sc as plsc

B = 128
N = 32768
KSEL = 3276
NV = N // 16
NW = 32
RPW = B // NW
UNROLL = 16


def _row_topk_sum(xv, yv, hist, tot, lanes, lane_off, ones_i, zeros_i,
                  prefetch):
    def merge_and_pick(kr):
        @plsc.parallel_loop(0, 16, unroll=2)
        def _(c):
            t = hist[pl.ds(c * 16, 16)]
            for l in range(1, 16):
                t = t + hist[pl.ds(l * 256 + c * 16, 16)]
            tot[pl.ds(c * 16, 16)] = jnp.flip(jnp.cumsum(jnp.flip(t)))

        krv = jnp.full((16,), kr, jnp.int32)
        ct = plsc.load_gather(tot, [lanes * 16])
        sc = jnp.flip(jnp.cumsum(jnp.flip(ct)))
        above = jnp.max(jnp.where(sc < krv, sc, 0))
        cstar = jnp.sum(jnp.where(sc >= krv, 1, 0)) - 1
        s_in = tot[pl.ds(cstar * 16, 16)] + above
        cnt_in = jnp.sum(jnp.where(s_in >= krv, 1, 0))
        bstar = cstar * 16 + cnt_in - 1
        over = jnp.maximum(above, jnp.max(jnp.where(s_in < krv, s_in, 0)))
        return bstar, over

    @plsc.parallel_loop(0, 256, unroll=UNROLL)
    def _(i):
        hist[pl.ds(i * 16, 16)] = zeros_i

    @plsc.parallel_loop(0, NV, unroll=UNROLL)
    def _(i):
        s0 = i * 16
        l = jnp.abs(xv[pl.ds(s0, 16)] - yv[pl.ds(s0, 16)])
        xv[pl.ds(s0, 16)] = l
        v = lax.bitcast_convert_type(l, jnp.int32)
        bucket = lax.shift_right_logical(v, 24)
        plsc.addupdate_scatter(hist, [lane_off + bucket], ones_i)

    bstar, over = merge_and_pick(jnp.int32(KSEL))
    P = lax.shift_left(bstar, 24)
    kr = jnp.int32(KSEL) - over

    pvec1 = jnp.full((16,), P, jnp.int32)

    @plsc.parallel_loop(0, 256, unroll=UNROLL)
    def _(i):
        hist[pl.ds(i * 16, 16)] = zeros_i

    @plsc.parallel_loop(0, NV, unroll=UNROLL)
    def _(i):
        v = lax.bitcast_convert_type(xv[pl.ds(i * 16, 16)], jnp.int32)
        bucket = lax.shift_right_logical(v, 16) & 255
        m = (v & jnp.int32(-(1 << 24))) == pvec1
        plsc.addupdate_scatter(hist, [lane_off + bucket], ones_i, mask=m)

    prefetch()

    bstar, over = merge_and_pick(kr)
    P = P | lax.shift_left(bstar, 16)
    kr = kr - over

    def pass_body(p, carries):
        P, kr = carries
        sh = 24 - 8 * p
        himask = lax.shift_left(jnp.int32(-1), sh + 8)
        pvec = jnp.full((16,), P, jnp.int32)
        shv = jnp.full((16,), sh, jnp.int32)
        hmv = jnp.full((16,), himask, jnp.int32)

        @plsc.parallel_loop(0, 256, unroll=UNROLL)
        def _(i):
            hist[pl.ds(i * 16, 16)] = zeros_i

        @plsc.parallel_loop(0, NV, unroll=UNROLL)
        def _(i):
            v = lax.bitcast_convert_type(xv[pl.ds(i * 16, 16)], jnp.int32)
            bucket = lax.shift_right_logical(v, shv) & 255
            m = (v & hmv) == pvec
            plsc.addupdate_scatter(hist, [lane_off + bucket],
                                   ones_i, mask=m)

        bstar, over = merge_and_pick(kr)
        return P | lax.shift_left(bstar, sh), kr - over

    P, kr = lax.fori_loop(2, 4, pass_body, (P, kr))

    tvec = lax.bitcast_convert_type(jnp.full((16,), P, jnp.int32), jnp.float32)
    tval = jnp.max(tvec)

    @plsc.parallel_loop(0, NV, unroll=UNROLL,
                        carry=(jnp.zeros((16,), jnp.float32), zeros_i))
    def fin_loop(i, c):
        acc, cnt = c
        v = xv[pl.ds(i * 16, 16)]
        gt = v > tvec
        acc = acc + jnp.where(gt, v, jnp.float32(0.0))
        cnt = cnt + jnp.where(gt, 1, 0)
        return acc, cnt
    acc, cnt = fin_loop
    return (jnp.sum(acc)
            + (jnp.int32(KSEL) - jnp.sum(cnt)).astype(jnp.float32) * tval)


def _tec_body(x_hbm, y_hbm, out_hbm, xv0, xv1, yv, hist, tot, outv,
              semx, semy):
    cid = lax.axis_index("c")
    sid = lax.axis_index("s")
    wid = sid * 2 + cid

    lanes = lax.iota(jnp.int32, 16)
    lane_off = lanes * 256
    ones_i = jnp.ones((16,), jnp.int32)
    zeros_i = jnp.zeros((16,), jnp.int32)

    def issue(rr, xbuf):
        pltpu.async_copy(x_hbm.at[rr], xbuf, semx)
        pltpu.async_copy(y_hbm.at[rr], yv, semy)

    def drain():
        pltpu.make_async_copy(x_hbm.at[0], xv0, semx).wait()
        pltpu.make_async_copy(y_hbm.at[0], yv, semy).wait()

    issue(wid * RPW, xv0)

    def pair_body(j, total):
        r = wid * RPW + 2 * j
        drain()
        s0 = _row_topk_sum(xv0, yv, hist, tot,
                           lanes, lane_off, ones_i, zeros_i,
                           lambda: issue(r + 1, xv1))
        drain()
        s1 = _row_topk_sum(xv1, yv, hist, tot,
                           lanes, lane_off, ones_i, zeros_i,
                           lambda: issue(jnp.minimum(r + 2, B - 1), xv0))
        return total + jnp.full((16,), s0 + s1, jnp.float32)

    total = lax.fori_loop(0, RPW // 2, pair_body, jnp.zeros((16,), jnp.float32))
    drain()

    outv[...] = total
    sync_out = pltpu.sync_copy(outv, out_hbm.at[wid])


@jax.jit
def _topk_partials(x, y):
    mesh = plsc.VectorSubcoreMesh(core_axis_name="c", subcore_axis_name="s")
    run = pl.kernel(
        _tec_body,
        out_type=jax.ShapeDtypeStruct((NW, 16), jnp.float32),
        mesh=mesh,
        compiler_params=pltpu.CompilerParams(needs_layout_passes=False),
        scratch_types=[
            pltpu.VMEM((N,), jnp.float32),
            pltpu.VMEM((N,), jnp.float32),
            pltpu.VMEM((N,), jnp.float32),
            pltpu.VMEM((4096,), jnp.int32),
            pltpu.VMEM((256,), jnp.int32),
            pltpu.VMEM((16,), jnp.float32),
            pltpu.SemaphoreType.DMA,
            pltpu.SemaphoreType.DMA,
        ],
    )
    return run(x, y)


def kernel(x, y):
    partials = _topk_partials(x, y)
    return jnp.sum(partials[:, 0]) / jnp.float32(B * KSEL)
